# baseline (speedup 1.0000x reference)
import jax
import jax.numpy as jnp
from jax.experimental import pallas as pl


def _prep_body(xa_ref, xb_ref, wa_ref, wb_ref, oa_ref, ob_ref):
    oa_ref[...] = jnp.dot(
        xa_ref[...].astype(jnp.bfloat16), wa_ref[...],
        preferred_element_type=jnp.float32,
    ).astype(jnp.bfloat16)
    ob_ref[...] = jnp.dot(
        xb_ref[...].astype(jnp.bfloat16), wb_ref[...],
        preferred_element_type=jnp.float32,
    ).astype(jnp.bfloat16)


def _l1_body(sa_ref, fa_ref, ya_ref, yb_ref, b1a_ref, b1b_ref,
             w2a_ref, w2b_ref, oa_ref, ob_ref):
    ha = jnp.dot(sa_ref[...].astype(jnp.bfloat16), ya_ref[...],
                 preferred_element_type=jnp.float32)
    ha = jnp.maximum(ha + b1a_ref[...], 0.0).astype(jnp.bfloat16)
    oa_ref[...] = jnp.dot(ha, w2a_ref[...],
                          preferred_element_type=jnp.float32
                          ).astype(jnp.bfloat16)
    hb = jnp.dot(fa_ref[...].astype(jnp.bfloat16), yb_ref[...],
                 preferred_element_type=jnp.float32)
    hb = jnp.maximum(hb + b1b_ref[...], 0.0).astype(jnp.bfloat16)
    ob_ref[...] = jnp.dot(hb, w2b_ref[...],
                          preferred_element_type=jnp.float32
                          ).astype(jnp.bfloat16)


def _l2_body(sa_ref, fa_ref, za_ref, zb_ref, b2a_ref, b2b_ref,
             mwa_ref, mwb_ref, mba_ref, mbb_ref, oa_ref, ob_ref):
    def head(adj_ref, z_ref, b2_ref, mw_ref, mb_ref, o_ref):
        s = jnp.dot(adj_ref[...].astype(jnp.bfloat16), z_ref[...],
                    preferred_element_type=jnp.float32) + b2_ref[...]
        logits = jnp.dot(s.astype(jnp.bfloat16), mw_ref[...],
                         preferred_element_type=jnp.float32) + mb_ref[...]
        mx = jnp.max(logits, axis=1, keepdims=True)
        sh = logits - mx
        o_ref[...] = sh - jnp.log(jnp.sum(jnp.exp(sh), axis=1, keepdims=True))

    head(sa_ref, za_ref, b2a_ref, mwa_ref, mba_ref, oa_ref)
    head(fa_ref, zb_ref, b2b_ref, mwb_ref, mbb_ref, ob_ref)


def _full(d1, d2):
    return pl.BlockSpec((d1, d2), lambda i: (0, 0))


def kernel(x_rumor, x_stance, sadj, fadj,
           s1_W1, s1_b1, s1_W2, s1_b2,
           s2_W1, s2_b1, s2_W2, s2_b2,
           c_W1, c_b1, c_W2, c_b2,
           mlp1_W, mlp1_b, mlp2_W, mlp2_b):
    n, nf = x_rumor.shape
    h1 = s1_W1.shape[1]
    h2 = s1_W2.shape[1]
    c1 = 2 * h1
    c2 = 2 * h2

    def w1cat(sW1, cW1):
        return jnp.concatenate([sW1, cW1], axis=1).astype(jnp.bfloat16)

    def w2blk(sW2, cW2):
        z = jnp.zeros((2 * h1, c2), jnp.bfloat16)
        z = z.at[:h1, :h2].set(sW2.astype(jnp.bfloat16))
        z = z.at[h1:, h2:].set(cW2.astype(jnp.bfloat16))
        return z

    def bcat(sb, cb):
        return jnp.concatenate([sb, cb]).reshape(1, -1)

    def mlppad(mW, mb):
        nout = mW.shape[1]
        mwp = jnp.zeros((mW.shape[0], 8), jnp.bfloat16)
        mwp = mwp.at[:, :nout].set(mW.astype(jnp.bfloat16))
        mbp = jnp.full((1, 8), -1e30, jnp.float32).at[0, :nout].set(mb)
        return mwp, mbp

    w1a, w1b = w1cat(s1_W1, c_W1), w1cat(s2_W1, c_W1)
    b1a, b1b = bcat(s1_b1, c_b1), bcat(s2_b1, c_b1)
    w2a, w2b = w2blk(s1_W2, c_W2), w2blk(s2_W2, c_W2)
    b2a, b2b = bcat(s1_b2, c_b2), bcat(s2_b2, c_b2)
    mwa, mba = mlppad(mlp1_W, mlp1_b)
    mwb, mbb = mlppad(mlp2_W, mlp2_b)

    bmp = min(2000, n)
    ya, yb = pl.pallas_call(
        _prep_body,
        grid=(n // bmp,),
        in_specs=[
            pl.BlockSpec((bmp, nf), lambda i: (i, 0)),
            pl.BlockSpec((bmp, nf), lambda i: (i, 0)),
            _full(nf, c1), _full(nf, c1),
        ],
        out_specs=[
            pl.BlockSpec((bmp, c1), lambda i: (i, 0)),
            pl.BlockSpec((bmp, c1), lambda i: (i, 0)),
        ],
        out_shape=[
            jax.ShapeDtypeStruct((n, c1), jnp.bfloat16),
            jax.ShapeDtypeStruct((n, c1), jnp.bfloat16),
        ],
    )(x_rumor, x_stance, w1a, w1b)

    bm = min(200, n)
    za, zb = pl.pallas_call(
        _l1_body,
        grid=(n // bm,),
        in_specs=[
            pl.BlockSpec((bm, n), lambda i: (i, 0)),
            pl.BlockSpec((bm, n), lambda i: (i, 0)),
            _full(n, c1), _full(n, c1),
            _full(1, c1), _full(1, c1),
            _full(c1, c2), _full(c1, c2),
        ],
        out_specs=[
            pl.BlockSpec((bm, c2), lambda i: (i, 0)),
            pl.BlockSpec((bm, c2), lambda i: (i, 0)),
        ],
        out_shape=[
            jax.ShapeDtypeStruct((n, c2), jnp.bfloat16),
            jax.ShapeDtypeStruct((n, c2), jnp.bfloat16),
        ],
    )(sadj, fadj, ya, yb, b1a, b1b, w2a, w2b)

    o1p, o2p = pl.pallas_call(
        _l2_body,
        grid=(n // bm,),
        in_specs=[
            pl.BlockSpec((bm, n), lambda i: (i, 0)),
            pl.BlockSpec((bm, n), lambda i: (i, 0)),
            _full(n, c2), _full(n, c2),
            _full(1, c2), _full(1, c2),
            _full(c2, 8), _full(c2, 8),
            _full(1, 8), _full(1, 8),
        ],
        out_specs=[
            pl.BlockSpec((bm, 8), lambda i: (i, 0)),
            pl.BlockSpec((bm, 8), lambda i: (i, 0)),
        ],
        out_shape=[
            jax.ShapeDtypeStruct((n, 8), jnp.float32),
            jax.ShapeDtypeStruct((n, 8), jnp.float32),
        ],
    )(sadj, fadj, za, zb, b2a, b2b, mwa, mwb, mba, mbb)

    return (o1p[:, : mlp1_W.shape[1]], o2p[:, : mlp2_W.shape[1]])


# speedup vs baseline: 1.0431x; 1.0431x over previous
import functools

import jax
import jax.numpy as jnp
from jax.experimental import pallas as pl
from jax.experimental.pallas import tpu as pltpu


def _xw_body(xa_ref, xb_ref, wa_ref, wb_ref, oa_ref, ob_ref):
    oa_ref[...] = jnp.dot(
        xa_ref[...].astype(jnp.bfloat16), wa_ref[...],
        preferred_element_type=jnp.float32,
    ).astype(jnp.bfloat16)
    ob_ref[...] = jnp.dot(
        xb_ref[...].astype(jnp.bfloat16), wb_ref[...],
        preferred_element_type=jnp.float32,
    ).astype(jnp.bfloat16)


def _mega_body(adj_ref, y_ref, b1_ref, w2_ref, b2_ref, mw_ref, mb_ref,
               o_ref, z_ref, *, bm):
    p = pl.program_id(0)
    i = pl.program_id(1)
    a = adj_ref[...].astype(jnp.bfloat16)

    @pl.when(p == 0)
    def _():
        acc = jnp.dot(a, y_ref[...], preferred_element_type=jnp.float32)
        h = jnp.maximum(acc + b1_ref[...], 0.0).astype(jnp.bfloat16)
        z_ref[pl.ds(i * bm, bm), :] = jnp.dot(
            h, w2_ref[...], preferred_element_type=jnp.float32
        ).astype(jnp.bfloat16)

    @pl.when(p == 1)
    def _():
        s = jnp.dot(a, z_ref[...],
                    preferred_element_type=jnp.float32) + b2_ref[...]
        logits = jnp.dot(
            s.astype(jnp.bfloat16), mw_ref[...],
            preferred_element_type=jnp.float32,
        ) + mb_ref[...]
        mx = jnp.max(logits, axis=1, keepdims=True)
        sh = logits - mx
        o_ref[...] = sh - jnp.log(jnp.sum(jnp.exp(sh), axis=1, keepdims=True))


def _branch(adj, y, b1, w2, b2, mw, mb, bm=400):
    n = adj.shape[0]
    bm = min(bm, n)
    c1 = y.shape[1]
    c2 = w2.shape[1]
    p = mw.shape[1]
    return pl.pallas_call(
        functools.partial(_mega_body, bm=bm),
        grid=(2, n // bm),
        in_specs=[
            pl.BlockSpec((bm, n), lambda ph, i: (i, 0)),
            pl.BlockSpec((n, c1), lambda ph, i: (0, 0)),
            pl.BlockSpec((1, c1), lambda ph, i: (0, 0)),
            pl.BlockSpec((c1, c2), lambda ph, i: (0, 0)),
            pl.BlockSpec((1, c2), lambda ph, i: (0, 0)),
            pl.BlockSpec((c2, p), lambda ph, i: (0, 0)),
            pl.BlockSpec((1, p), lambda ph, i: (0, 0)),
        ],
        out_specs=pl.BlockSpec((bm, p), lambda ph, i: (i, 0)),
        out_shape=jax.ShapeDtypeStruct((n, p), jnp.float32),
        scratch_shapes=[pltpu.VMEM((n, c2), jnp.bfloat16)],
    )(adj, y, b1, w2, b2, mw, mb)


def kernel(x_rumor, x_stance, sadj, fadj,
           s1_W1, s1_b1, s1_W2, s1_b2,
           s2_W1, s2_b1, s2_W2, s2_b2,
           c_W1, c_b1, c_W2, c_b2,
           mlp1_W, mlp1_b, mlp2_W, mlp2_b):
    n, nf = x_rumor.shape
    h1 = s1_W1.shape[1]
    h2 = s1_W2.shape[1]
    c1 = 2 * h1
    c2 = 2 * h2

    def w1cat(sW1, cW1):
        return jnp.concatenate([sW1, cW1], axis=1).astype(jnp.bfloat16)

    def w2blk(sW2, cW2):
        z = jnp.zeros((c1, c2), jnp.bfloat16)
        z = z.at[:h1, :h2].set(sW2.astype(jnp.bfloat16))
        z = z.at[h1:, h2:].set(cW2.astype(jnp.bfloat16))
        return z

    def bcat(sb, cb):
        return jnp.concatenate([sb, cb]).reshape(1, -1)

    def mlppad(mW, mb):
        nout = mW.shape[1]
        mwp = jnp.zeros((mW.shape[0], 8), jnp.bfloat16).at[:, :nout].set(
            mW.astype(jnp.bfloat16))
        mbp = jnp.full((1, 8), -1e30, jnp.float32).at[0, :nout].set(mb)
        return mwp, mbp

    w1a, w1b = w1cat(s1_W1, c_W1), w1cat(s2_W1, c_W1)
    b1a, b1b = bcat(s1_b1, c_b1), bcat(s2_b1, c_b1)
    w2a, w2b = w2blk(s1_W2, c_W2), w2blk(s2_W2, c_W2)
    b2a, b2b = bcat(s1_b2, c_b2), bcat(s2_b2, c_b2)
    mwa, mba = mlppad(mlp1_W, mlp1_b)
    mwb, mbb = mlppad(mlp2_W, mlp2_b)

    bmp = min(2000, n)
    ya, yb = pl.pallas_call(
        _xw_body,
        grid=(n // bmp,),
        in_specs=[
            pl.BlockSpec((bmp, nf), lambda i: (i, 0)),
            pl.BlockSpec((bmp, nf), lambda i: (i, 0)),
            pl.BlockSpec((nf, c1), lambda i: (0, 0)),
            pl.BlockSpec((nf, c1), lambda i: (0, 0)),
        ],
        out_specs=[
            pl.BlockSpec((bmp, c1), lambda i: (i, 0)),
            pl.BlockSpec((bmp, c1), lambda i: (i, 0)),
        ],
        out_shape=[
            jax.ShapeDtypeStruct((n, c1), jnp.bfloat16),
            jax.ShapeDtypeStruct((n, c1), jnp.bfloat16),
        ],
    )(x_rumor, x_stance, w1a, w1b)

    o1p = _branch(sadj, ya, b1a, w2a, b2a, mwa, mba)
    o2p = _branch(fadj, yb, b1b, w2b, b2b, mwb, mbb)
    return (o1p[:, : mlp1_W.shape[1]], o2p[:, : mlp2_W.shape[1]])


# speedup vs baseline: 1.0498x; 1.0064x over previous
import functools

import jax
import jax.numpy as jnp
from jax.experimental import pallas as pl
from jax.experimental.pallas import tpu as pltpu


def _mega_body(adj_ref, x_ref, w1_ref, b1_ref, w2_ref, b2_ref, mw_ref,
               mb_ref, o_ref, y_ref, z_ref, *, bm):
    p = pl.program_id(0)
    i = pl.program_id(1)

    @pl.when((p == 0) & (i == 0))
    def _():
        y_ref[...] = jnp.dot(
            x_ref[...].astype(jnp.bfloat16), w1_ref[...],
            preferred_element_type=jnp.float32,
        ).astype(jnp.bfloat16)

    a = adj_ref[...].astype(jnp.bfloat16)

    @pl.when(p == 0)
    def _():
        acc = jnp.dot(a, y_ref[...], preferred_element_type=jnp.float32)
        h = jnp.maximum(acc + b1_ref[...], 0.0).astype(jnp.bfloat16)
        z_ref[pl.ds(i * bm, bm), :] = jnp.dot(
            h, w2_ref[...], preferred_element_type=jnp.float32
        ).astype(jnp.bfloat16)

    @pl.when(p == 1)
    def _():
        s = jnp.dot(a, z_ref[...],
                    preferred_element_type=jnp.float32) + b2_ref[...]
        logits = jnp.dot(
            s.astype(jnp.bfloat16), mw_ref[...],
            preferred_element_type=jnp.float32,
        ) + mb_ref[...]
        mx = jnp.max(logits, axis=1, keepdims=True)
        sh = logits - mx
        o_ref[...] = sh - jnp.log(jnp.sum(jnp.exp(sh), axis=1, keepdims=True))


def _branch(adj, x, w1, b1, w2, b2, mw, mb, bm=400):
    n = adj.shape[0]
    bm = min(bm, n)
    nf = x.shape[1]
    c1 = w1.shape[1]
    c2 = w2.shape[1]
    p = mw.shape[1]
    return pl.pallas_call(
        functools.partial(_mega_body, bm=bm),
        grid=(2, n // bm),
        in_specs=[
            pl.BlockSpec((bm, n), lambda ph, i: (i, 0)),
            pl.BlockSpec((n, nf), lambda ph, i: (0, 0)),
            pl.BlockSpec((nf, c1), lambda ph, i: (0, 0)),
            pl.BlockSpec((1, c1), lambda ph, i: (0, 0)),
            pl.BlockSpec((c1, c2), lambda ph, i: (0, 0)),
            pl.BlockSpec((1, c2), lambda ph, i: (0, 0)),
            pl.BlockSpec((c2, p), lambda ph, i: (0, 0)),
            pl.BlockSpec((1, p), lambda ph, i: (0, 0)),
        ],
        out_specs=pl.BlockSpec((bm, p), lambda ph, i: (i, 0)),
        out_shape=jax.ShapeDtypeStruct((n, p), jnp.float32),
        scratch_shapes=[
            pltpu.VMEM((n, c1), jnp.bfloat16),
            pltpu.VMEM((n, c2), jnp.bfloat16),
        ],
    )(adj, x, w1, b1, w2, b2, mw, mb)


def kernel(x_rumor, x_stance, sadj, fadj,
           s1_W1, s1_b1, s1_W2, s1_b2,
           s2_W1, s2_b1, s2_W2, s2_b2,
           c_W1, c_b1, c_W2, c_b2,
           mlp1_W, mlp1_b, mlp2_W, mlp2_b):
    h1 = s1_W1.shape[1]
    h2 = s1_W2.shape[1]
    c1 = 2 * h1
    c2 = 2 * h2

    def w1cat(sW1, cW1):
        return jnp.concatenate([sW1, cW1], axis=1).astype(jnp.bfloat16)

    def w2blk(sW2, cW2):
        z = jnp.zeros((c1, c2), jnp.bfloat16)
        z = z.at[:h1, :h2].set(sW2.astype(jnp.bfloat16))
        z = z.at[h1:, h2:].set(cW2.astype(jnp.bfloat16))
        return z

    def bcat(sb, cb):
        return jnp.concatenate([sb, cb]).reshape(1, -1)

    def mlppad(mW, mb):
        nout = mW.shape[1]
        mwp = jnp.zeros((mW.shape[0], 8), jnp.bfloat16).at[:, :nout].set(
            mW.astype(jnp.bfloat16))
        mbp = jnp.full((1, 8), -1e30, jnp.float32).at[0, :nout].set(mb)
        return mwp, mbp

    w1a, w1b = w1cat(s1_W1, c_W1), w1cat(s2_W1, c_W1)
    b1a, b1b = bcat(s1_b1, c_b1), bcat(s2_b1, c_b1)
    w2a, w2b = w2blk(s1_W2, c_W2), w2blk(s2_W2, c_W2)
    b2a, b2b = bcat(s1_b2, c_b2), bcat(s2_b2, c_b2)
    mwa, mba = mlppad(mlp1_W, mlp1_b)
    mwb, mbb = mlppad(mlp2_W, mlp2_b)

    o1p = _branch(sadj, x_rumor, w1a, b1a, w2a, b2a, mwa, mba)
    o2p = _branch(fadj, x_stance, w1b, b1b, w2b, b2b, mwb, mbb)
    return (o1p[:, : mlp1_W.shape[1]], o2p[:, : mlp2_W.shape[1]])
